# Initial kernel scaffold; baseline (speedup 1.0000x reference)
import jax
import jax.numpy as jnp
from jax.experimental import pallas as pl


def kernel(x, W_gate, b_gate):
    raise NotImplementedError("write your pallas kernel here")


# speedup vs baseline: 1.0683x; 1.0683x over previous
import jax
import jax.numpy as jnp
from jax.experimental import pallas as pl
from jax.experimental.pallas import tpu as pltpu

TOPK = 8
E = 64
D = 2048
N = 16384
BLK = 512
GRID = N // BLK


def _router_body(x_ref, w_ref, b_ref, idx_ref, val_ref, psum_ref, acc_ref):
    logits = jnp.dot(x_ref[...], w_ref[...], preferred_element_type=jnp.float32)
    logits = logits + b_ref[...]
    m = jnp.max(logits, axis=-1, keepdims=True)
    ex = jnp.exp(logits - m)
    scores = ex / jnp.sum(ex, axis=-1, keepdims=True)

    lane = jax.lax.broadcasted_iota(jnp.int32, scores.shape, 1)
    work = scores
    vals = []
    idxs = []
    for _ in range(TOPK):
        mk = jnp.max(work, axis=-1, keepdims=True)
        ik = jnp.min(jnp.where(work == mk, lane, E), axis=-1, keepdims=True)
        vals.append(mk)
        idxs.append(ik)
        work = jnp.where(lane == ik, -jnp.inf, work)
    v = jnp.concatenate(vals, axis=-1)
    i = jnp.concatenate(idxs, axis=-1)
    idx_ref[...] = i
    val_ref[...] = v

    @pl.when(pl.program_id(0) == 0)
    def _():
        acc_ref[0, 0] = 0.0

    acc_ref[0, 0] += jnp.sum(v)

    @pl.when(pl.program_id(0) == GRID - 1)
    def _():
        psum_ref[0, 0] = acc_ref[0, 0]


def _norm_body(val_ref, psum_ref, out_ref):
    out_ref[...] = val_ref[...] * (1.0 / psum_ref[0, 0])


def kernel(x, W_gate, b_gate):
    b2 = b_gate.reshape(1, E)
    idx, val, psum = pl.pallas_call(
        _router_body,
        grid=(GRID,),
        in_specs=[
            pl.BlockSpec((BLK, D), lambda i: (i, 0)),
            pl.BlockSpec((D, E), lambda i: (0, 0)),
            pl.BlockSpec((1, E), lambda i: (0, 0)),
        ],
        out_specs=[
            pl.BlockSpec((BLK, TOPK), lambda i: (i, 0)),
            pl.BlockSpec((BLK, TOPK), lambda i: (i, 0)),
            pl.BlockSpec(memory_space=pltpu.SMEM),
        ],
        out_shape=[
            jax.ShapeDtypeStruct((N, TOPK), jnp.int32),
            jax.ShapeDtypeStruct((N, TOPK), jnp.float32),
            jax.ShapeDtypeStruct((1, 1), jnp.float32),
        ],
        scratch_shapes=[pltpu.SMEM((1, 1), jnp.float32)],
    )(x, W_gate, b2)

    weights = pl.pallas_call(
        _norm_body,
        in_specs=[
            pl.BlockSpec((N, TOPK), lambda: (0, 0)),
            pl.BlockSpec(memory_space=pltpu.SMEM),
        ],
        out_shape=jax.ShapeDtypeStruct((N, TOPK), jnp.float32),
    )(val, psum)
    return idx.reshape(-1), weights


# speedup vs baseline: 1.3055x; 1.2221x over previous
import jax
import jax.numpy as jnp
from jax.experimental import pallas as pl
from jax.experimental.pallas import tpu as pltpu

TOPK = 8
E = 64
D = 2048
N = 16384
BLK = 512
GRID = N // BLK


def _router_body(x_ref, w_ref, b_ref, idx_ref, val_ref, psum_ref, acc_ref):
    logits = jnp.dot(x_ref[...], w_ref[...], preferred_element_type=jnp.float32)
    logits = logits + b_ref[...]
    m = jnp.max(logits, axis=-1, keepdims=True)
    ex = jnp.exp(logits - m)
    scores = ex / jnp.sum(ex, axis=-1, keepdims=True)

    lane = jax.lax.broadcasted_iota(jnp.int32, scores.shape, 1).astype(jnp.float32)
    work = scores
    vals = []
    idxs = []
    for _ in range(TOPK):
        mk = jnp.max(work, axis=-1, keepdims=True)
        ik = jnp.min(jnp.where(work == mk, lane, float(E)), axis=-1, keepdims=True)
        vals.append(mk)
        idxs.append(ik)
        work = jnp.where(lane == ik, -jnp.inf, work)
    v = jnp.concatenate(vals, axis=-1)
    i = jnp.concatenate(idxs, axis=-1)
    idx_ref[...] = i.astype(jnp.int32)
    val_ref[...] = v

    @pl.when(pl.program_id(0) == 0)
    def _():
        acc_ref[0, 0] = 0.0

    acc_ref[0, 0] += jnp.sum(v)

    @pl.when(pl.program_id(0) == GRID - 1)
    def _():
        psum_ref[0, 0] = acc_ref[0, 0]


def _norm_body(val_ref, psum_ref, out_ref):
    out_ref[...] = val_ref[...] * (1.0 / psum_ref[0, 0])


def kernel(x, W_gate, b_gate):
    b2 = b_gate.reshape(1, E)
    idx, val, psum = pl.pallas_call(
        _router_body,
        grid=(GRID,),
        in_specs=[
            pl.BlockSpec((BLK, D), lambda i: (i, 0)),
            pl.BlockSpec((D, E), lambda i: (0, 0)),
            pl.BlockSpec((1, E), lambda i: (0, 0)),
        ],
        out_specs=[
            pl.BlockSpec((BLK, TOPK), lambda i: (i, 0)),
            pl.BlockSpec((BLK, TOPK), lambda i: (i, 0)),
            pl.BlockSpec(memory_space=pltpu.SMEM),
        ],
        out_shape=[
            jax.ShapeDtypeStruct((N, TOPK), jnp.int32),
            jax.ShapeDtypeStruct((N, TOPK), jnp.float32),
            jax.ShapeDtypeStruct((1, 1), jnp.float32),
        ],
        scratch_shapes=[pltpu.SMEM((1, 1), jnp.float32)],
    )(x, W_gate, b2)

    weights = pl.pallas_call(
        _norm_body,
        in_specs=[
            pl.BlockSpec((N, TOPK), lambda: (0, 0)),
            pl.BlockSpec(memory_space=pltpu.SMEM),
        ],
        out_shape=jax.ShapeDtypeStruct((N, TOPK), jnp.float32),
    )(val, psum)
    return idx.reshape(-1), weights


# speedup vs baseline: 1.5228x; 1.1665x over previous
import jax
import jax.numpy as jnp
from jax.experimental import pallas as pl
from jax.experimental.pallas import tpu as pltpu

TOPK = 8
E = 64
D = 2048
N = 16384
BLK = 512
GRID = N // BLK


def _router_body(x_ref, w_ref, b_ref, idx_ref, val_ref, psum_ref, acc_ref):
    logits = jnp.dot(x_ref[...], w_ref[...], preferred_element_type=jnp.float32)
    logits = logits + b_ref[...]
    lt = logits.T
    m = jnp.max(lt, axis=0, keepdims=True)
    ex = jnp.exp(lt - m)
    scores = ex / jnp.sum(ex, axis=0, keepdims=True)

    lane = jax.lax.broadcasted_iota(jnp.int32, scores.shape, 0).astype(jnp.float32)
    work = scores
    vals = []
    idxs = []
    for _ in range(TOPK):
        mk = jnp.max(work, axis=0, keepdims=True)
        ik = jnp.min(jnp.where(work == mk, lane, float(E)), axis=0, keepdims=True)
        vals.append(mk)
        idxs.append(ik)
        work = jnp.where(lane == ik, -jnp.inf, work)
    v = jnp.concatenate(vals, axis=0)
    i = jnp.concatenate(idxs, axis=0)
    idx_ref[...] = i.T.astype(jnp.int32)
    val_ref[...] = v.T

    @pl.when(pl.program_id(0) == 0)
    def _():
        acc_ref[0, 0] = 0.0

    acc_ref[0, 0] += jnp.sum(v)

    @pl.when(pl.program_id(0) == GRID - 1)
    def _():
        psum_ref[0, 0] = acc_ref[0, 0]


def _norm_body(val_ref, psum_ref, out_ref):
    out_ref[...] = val_ref[...] * (1.0 / psum_ref[0, 0])


def kernel(x, W_gate, b_gate):
    b2 = b_gate.reshape(1, E)
    idx, val, psum = pl.pallas_call(
        _router_body,
        grid=(GRID,),
        in_specs=[
            pl.BlockSpec((BLK, D), lambda i: (i, 0)),
            pl.BlockSpec((D, E), lambda i: (0, 0)),
            pl.BlockSpec((1, E), lambda i: (0, 0)),
        ],
        out_specs=[
            pl.BlockSpec((BLK, TOPK), lambda i: (i, 0)),
            pl.BlockSpec((BLK, TOPK), lambda i: (i, 0)),
            pl.BlockSpec(memory_space=pltpu.SMEM),
        ],
        out_shape=[
            jax.ShapeDtypeStruct((N, TOPK), jnp.int32),
            jax.ShapeDtypeStruct((N, TOPK), jnp.float32),
            jax.ShapeDtypeStruct((1, 1), jnp.float32),
        ],
        scratch_shapes=[pltpu.SMEM((1, 1), jnp.float32)],
    )(x, W_gate, b2)

    weights = pl.pallas_call(
        _norm_body,
        in_specs=[
            pl.BlockSpec((N, TOPK), lambda: (0, 0)),
            pl.BlockSpec(memory_space=pltpu.SMEM),
        ],
        out_shape=jax.ShapeDtypeStruct((N, TOPK), jnp.float32),
    )(val, psum)
    return idx.reshape(-1), weights
